# baseline (speedup 1.0000x reference)
import functools

import jax
import jax.numpy as jnp
from jax import lax
from jax.experimental import pallas as pl
from jax.experimental.pallas import tpu as pltpu
from jax.experimental.pallas import tpu_sc as plsc

E = 64
K = 2
H = 768
F = 512
S = 2048

T = 64
NT = S * K // T + E
R = NT * T

SC_CORES = 2
SC_SUBCORES = 16
NW = SC_CORES * SC_SUBCORES


def _routing_body(logits_ref, idx_ref, wts_ref):
    logits = logits_ref[...]
    m = jnp.max(logits, axis=1, keepdims=True)
    ex = jnp.exp(logits - m)
    probs = ex / jnp.sum(ex, axis=1, keepdims=True)
    iota = lax.broadcasted_iota(jnp.int32, (S, E), 1)
    m1 = jnp.max(probs, axis=1, keepdims=True)
    i1 = jnp.min(jnp.where(probs == m1, iota, E), axis=1, keepdims=True)
    probs2 = jnp.where(iota == i1, -1.0, probs)
    m2 = jnp.max(probs2, axis=1, keepdims=True)
    i2 = jnp.min(jnp.where(probs2 == m2, iota, E), axis=1, keepdims=True)
    ssum = m1 + m2
    idx_ref[:, 0:1] = i1
    idx_ref[:, 1:2] = i2
    wts_ref[:, 0:1] = m1 / ssum
    wts_ref[:, 1:2] = m2 / ssum


def _routing(router_logits):
    return pl.pallas_call(
        _routing_body,
        out_shape=(
            jax.ShapeDtypeStruct((S, 8), jnp.int32),
            jax.ShapeDtypeStruct((S, 8), jnp.float32),
        ),
    )(router_logits.astype(jnp.float32))


def _plan(idx, wts):
    flat_e = jnp.stack([idx[:, 0], idx[:, 1]], axis=1).reshape(-1)
    flat_w = jnp.stack([wts[:, 0], wts[:, 1]], axis=1).reshape(-1)
    onehot = (flat_e[:, None] == jnp.arange(E, dtype=jnp.int32)[None, :])
    ohf = onehot.astype(jnp.float32)
    cum = jnp.cumsum(ohf, axis=0)
    rank = (jnp.sum(ohf * cum, axis=1) - 1.0).astype(jnp.int32)
    counts = cum[-1].astype(jnp.int32)
    tiles_per_e = (counts + T - 1) // T
    tile_csum = jnp.cumsum(tiles_per_e)
    toff = tile_csum - tiles_per_e
    toff_e = (ohf @ toff.astype(jnp.float32)).astype(jnp.int32)
    padded_pos = toff_e * T + rank

    tok_ids = (jnp.arange(S * K, dtype=jnp.int32) // K)
    row_tok = jnp.zeros((R,), jnp.int32).at[padded_pos].set(tok_ids)
    row_w = jnp.zeros((R,), jnp.float32).at[padded_pos].set(flat_w)
    p0 = padded_pos[0::2]
    p1 = padded_pos[1::2]

    total_tiles = tile_csum[-1]
    tids = jnp.arange(NT, dtype=jnp.int32)
    raw_te = jnp.searchsorted(tile_csum, tids, side="right").astype(jnp.int32)
    last_used = jnp.max(jnp.where(counts > 0, jnp.arange(E), -1)).astype(jnp.int32)
    used = tids < total_tiles
    tile_expert = jnp.where(used, jnp.minimum(raw_te, E - 1), last_used)
    cnt = counts[tile_expert].astype(jnp.int32) - (tids - toff[tile_expert].astype(jnp.int32)) * T
    tile_count = jnp.where(used, jnp.clip(cnt, 0, T), 0).astype(jnp.int32)

    rw_brd = jnp.broadcast_to(row_w.reshape(NT, T, 1), (NT, T, 128))
    tile_blk = jnp.where(used, tids, jnp.maximum(total_tiles - 1, 0)).astype(jnp.int32)
    n_rows = jnp.broadcast_to((total_tiles * T).astype(jnp.int32), (16,))
    return row_tok, rw_brd, p0, p1, tile_expert, tile_count, tile_blk, n_rows


_G_CH = 128
_G_NCH = R // _G_CH // NW


def _sc_mesh():
    return plsc.VectorSubcoreMesh(core_axis_name="c", subcore_axis_name="s")


_G_NS = 4
_G_SB = _G_CH // _G_NS


def _sc_gather_body(hs_hbm, tok_hbm, nrow_hbm, out_hbm, nrow_v, idx_v,
                    b0, b1, b2, b3, s0, s1, s2, s3):
    wid = lax.axis_index("s") * SC_CORES + lax.axis_index("c")
    pltpu.sync_copy(nrow_hbm, nrow_v)
    n = nrow_v[...][0]
    bufs = (b0, b1, b2, b3)
    sems = (s0, s1, s2, s3)
    for c in range(_G_NCH):
        o = (wid + NW * c) * _G_CH

        @pl.when(o < n)
        def _():
            pltpu.sync_copy(tok_hbm.at[pl.ds(o, _G_CH)], idx_v)
            cps = [
                pltpu.async_copy(
                    hs_hbm.at[idx_v.at[pl.ds(b * _G_SB, _G_SB)]],
                    bufs[b], sems[b])
                for b in range(_G_NS)
            ]
            for b in range(_G_NS):
                cps[b].wait()
                pltpu.sync_copy(bufs[b],
                                out_hbm.at[pl.ds(o + b * _G_SB, _G_SB)])


def _sc_gather(hidden_states, row_tok, n_rows):
    return pl.kernel(
        _sc_gather_body,
        out_type=jax.ShapeDtypeStruct((R, H), jnp.float32),
        mesh=_sc_mesh(),
        scratch_types=[
            pltpu.VMEM((16,), jnp.int32),
            pltpu.VMEM((_G_CH,), jnp.int32),
            pltpu.VMEM((_G_SB, H), jnp.float32),
            pltpu.VMEM((_G_SB, H), jnp.float32),
            pltpu.VMEM((_G_SB, H), jnp.float32),
            pltpu.VMEM((_G_SB, H), jnp.float32),
            pltpu.SemaphoreType.DMA,
            pltpu.SemaphoreType.DMA,
            pltpu.SemaphoreType.DMA,
            pltpu.SemaphoreType.DMA,
        ],
    )(hidden_states, row_tok, n_rows)


def _ffn_body(te_ref, tc_ref, tb_ref, rt_ref, hs_ref, w1_ref, w2_ref, rw_ref,
              y_ref, x2_s):
    i = pl.program_id(0)
    par = lax.rem(i, 2)
    nxt = jnp.minimum(i + 1, NT - 1)

    @pl.when(i == 0)
    def _():
        for j in range(T):
            t0 = rt_ref[j]
            x2_s[0, pl.ds(j, 1), :] = hs_ref[pl.ds(t0, 1), :]

    nb = 1 - par
    for j in range(T):
        t = rt_ref[nxt * T + j]
        x2_s[nb, pl.ds(j, 1), :] = hs_ref[pl.ds(t, 1), :]

    x = x2_s[par]
    gu = lax.dot_general(x, w1_ref[0], (((1,), (1,)), ((), ())),
                         preferred_element_type=jnp.float32)
    g = gu[:, :F]
    u = gu[:, F:]
    h = g * (1.0 / (1.0 + jnp.exp(-g))) * u
    y = lax.dot_general(h, w2_ref[0], (((1,), (1,)), ((), ())),
                        preferred_element_type=jnp.float32)
    yw = y * rw_ref[0, :, 0:1]

    @pl.when(tc_ref[i] > 0)
    def _():
        y_ref[...] = yw


def _ffn(hidden_states, w1, w2, rw_brd, tile_expert, tile_count, tile_blk,
         row_tok):
    grid_spec = pltpu.PrefetchScalarGridSpec(
        num_scalar_prefetch=4,
        grid=(NT,),
        in_specs=[
            pl.BlockSpec((S, H), lambda i, te, tc, tb, rt: (0, 0)),
            pl.BlockSpec((1, 2 * F, H), lambda i, te, tc, tb, rt: (te[i], 0, 0)),
            pl.BlockSpec((1, H, F), lambda i, te, tc, tb, rt: (te[i], 0, 0)),
            pl.BlockSpec((1, T, 128), lambda i, te, tc, tb, rt: (tb[i], 0, 0)),
        ],
        out_specs=pl.BlockSpec((T, H), lambda i, te, tc, tb, rt: (tb[i], 0)),
        scratch_shapes=[pltpu.VMEM((2, T, H), jnp.float32)],
    )
    return pl.pallas_call(
        _ffn_body,
        grid_spec=grid_spec,
        out_shape=jax.ShapeDtypeStruct((R, H), jnp.float32),
        compiler_params=pltpu.CompilerParams(
            dimension_semantics=("arbitrary",)),
    )(tile_expert, tile_count, tile_blk, row_tok, hidden_states, w1, w2,
      rw_brd)


_C_PER_W = S // NW
_C_HALF = _C_PER_W // 2
_LANES = 16


def _sc_combine_body(y_hbm, p0_hbm, p1_hbm, out_hbm, i0_v, i1_v, r0_v, r1_v,
                     s0, s1, s2, s3):
    wid = lax.axis_index("s") * SC_CORES + lax.axis_index("c")
    o = wid * _C_PER_W
    pltpu.sync_copy(p0_hbm.at[pl.ds(o, _C_PER_W)], i0_v)
    pltpu.sync_copy(p1_hbm.at[pl.ds(o, _C_PER_W)], i1_v)
    cps = [
        pltpu.async_copy(y_hbm.at[i0_v.at[pl.ds(0, _C_HALF)]],
                         r0_v.at[pl.ds(0, _C_HALF)], s0),
        pltpu.async_copy(y_hbm.at[i0_v.at[pl.ds(_C_HALF, _C_HALF)]],
                         r0_v.at[pl.ds(_C_HALF, _C_HALF)], s1),
        pltpu.async_copy(y_hbm.at[i1_v.at[pl.ds(0, _C_HALF)]],
                         r1_v.at[pl.ds(0, _C_HALF)], s2),
        pltpu.async_copy(y_hbm.at[i1_v.at[pl.ds(_C_HALF, _C_HALF)]],
                         r1_v.at[pl.ds(_C_HALF, _C_HALF)], s3),
    ]
    for cp in cps:
        cp.wait()

    def row_add(i, _):
        def lane_add(j, _):
            sl = pl.ds(j * _LANES, _LANES)
            r0_v[i, sl] = r0_v[i, sl] + r1_v[i, sl]
            return 0
        return lax.fori_loop(0, H // _LANES, lane_add, 0)

    lax.fori_loop(0, _C_PER_W, row_add, 0)
    pltpu.sync_copy(r0_v, out_hbm.at[pl.ds(o, _C_PER_W)])


def _sc_combine(y, p0, p1):
    return pl.kernel(
        _sc_combine_body,
        out_type=jax.ShapeDtypeStruct((S, H), jnp.float32),
        mesh=_sc_mesh(),
        scratch_types=[
            pltpu.VMEM((_C_PER_W,), jnp.int32),
            pltpu.VMEM((_C_PER_W,), jnp.int32),
            pltpu.VMEM((_C_PER_W, H), jnp.float32),
            pltpu.VMEM((_C_PER_W, H), jnp.float32),
            pltpu.SemaphoreType.DMA,
            pltpu.SemaphoreType.DMA,
            pltpu.SemaphoreType.DMA,
            pltpu.SemaphoreType.DMA,
        ],
    )(y, p0, p1)


def kernel(hidden_states, router_logits, w1, w2):
    idx, wts = _routing(router_logits)
    (row_tok, rw_brd, p0, p1, tile_expert, tile_count, tile_blk,
     n_rows) = _plan(idx, wts)
    y = _ffn(hidden_states, w1, w2, rw_brd, tile_expert, tile_count,
             tile_blk, row_tok)
    return _sc_combine(y, p0, p1)


# speedup vs baseline: 1.1073x; 1.1073x over previous
import functools

import jax
import jax.numpy as jnp
from jax import lax
from jax.experimental import pallas as pl
from jax.experimental.pallas import tpu as pltpu
from jax.experimental.pallas import tpu_sc as plsc

E = 64
K = 2
H = 768
F = 512
S = 2048

T = 64
NT = S * K // T + E
R = NT * T

SC_CORES = 2
SC_SUBCORES = 16
NW = SC_CORES * SC_SUBCORES


def _routing_body(logits_ref, idx_ref, wts_ref):
    logits = logits_ref[...]
    m = jnp.max(logits, axis=1, keepdims=True)
    ex = jnp.exp(logits - m)
    probs = ex / jnp.sum(ex, axis=1, keepdims=True)
    iota = lax.broadcasted_iota(jnp.int32, (S, E), 1)
    m1 = jnp.max(probs, axis=1, keepdims=True)
    i1 = jnp.min(jnp.where(probs == m1, iota, E), axis=1, keepdims=True)
    probs2 = jnp.where(iota == i1, -1.0, probs)
    m2 = jnp.max(probs2, axis=1, keepdims=True)
    i2 = jnp.min(jnp.where(probs2 == m2, iota, E), axis=1, keepdims=True)
    ssum = m1 + m2
    idx_ref[:, 0:1] = i1
    idx_ref[:, 1:2] = i2
    wts_ref[:, 0:1] = m1 / ssum
    wts_ref[:, 1:2] = m2 / ssum


def _routing(router_logits):
    return pl.pallas_call(
        _routing_body,
        out_shape=(
            jax.ShapeDtypeStruct((S, 8), jnp.int32),
            jax.ShapeDtypeStruct((S, 8), jnp.float32),
        ),
    )(router_logits.astype(jnp.float32))


def _plan(idx, wts):
    flat_e = jnp.stack([idx[:, 0], idx[:, 1]], axis=1).reshape(-1)
    flat_w = jnp.stack([wts[:, 0], wts[:, 1]], axis=1).reshape(-1)
    onehot = (flat_e[:, None] == jnp.arange(E, dtype=jnp.int32)[None, :])
    ohf = onehot.astype(jnp.float32)
    cum = jnp.cumsum(ohf, axis=0)
    rank = (jnp.sum(ohf * cum, axis=1) - 1.0).astype(jnp.int32)
    counts = cum[-1].astype(jnp.int32)
    tiles_per_e = (counts + T - 1) // T
    tile_csum = jnp.cumsum(tiles_per_e)
    toff = tile_csum - tiles_per_e
    toff_e = (ohf @ toff.astype(jnp.float32)).astype(jnp.int32)
    padded_pos = toff_e * T + rank

    tok_ids = (jnp.arange(S * K, dtype=jnp.int32) // K)
    row_tok = jnp.zeros((R,), jnp.int32).at[padded_pos].set(tok_ids)
    row_w = jnp.zeros((R,), jnp.float32).at[padded_pos].set(flat_w)
    p0 = padded_pos[0::2]
    p1 = padded_pos[1::2]

    total_tiles = tile_csum[-1]
    tids = jnp.arange(NT, dtype=jnp.int32)
    raw_te = jnp.searchsorted(tile_csum, tids, side="right").astype(jnp.int32)
    last_used = jnp.max(jnp.where(counts > 0, jnp.arange(E), -1)).astype(jnp.int32)
    used = tids < total_tiles
    tile_expert = jnp.where(used, jnp.minimum(raw_te, E - 1), last_used)
    cnt = counts[tile_expert].astype(jnp.int32) - (tids - toff[tile_expert].astype(jnp.int32)) * T
    tile_count = jnp.where(used, jnp.clip(cnt, 0, T), 0).astype(jnp.int32)

    rw_brd = jnp.broadcast_to(row_w.reshape(NT, T, 1), (NT, T, 128))
    tile_blk = jnp.where(used, tids, jnp.maximum(total_tiles - 1, 0)).astype(jnp.int32)
    n_rows = jnp.broadcast_to((total_tiles * T).astype(jnp.int32), (16,))

    used_e = counts > 0
    n_used = jnp.sum(used_e.astype(jnp.int32))
    rank_e = jnp.cumsum(used_e.astype(jnp.int32)) - 1
    eidx_t = rank_e[tile_expert]
    new_e = (used & (tids == toff[tile_expert].astype(jnp.int32))).astype(jnp.int32)
    par_t = (eidx_t % 2).astype(jnp.int32)
    exp_of_rank = jnp.searchsorted(
        jnp.cumsum(used_e.astype(jnp.int32)),
        jnp.arange(1, E + 1, dtype=jnp.int32), side="left").astype(jnp.int32)
    nxt_src = jnp.clip(exp_of_rank[jnp.clip(eidx_t + 1, 0, E - 1)], 0, E - 1)
    nxt_flag = (new_e.astype(bool) & (eidx_t + 1 < n_used)).astype(jnp.int32)

    return (row_tok, rw_brd, p0, p1, tile_expert, tile_count, tile_blk,
            n_rows, new_e, par_t, nxt_src, nxt_flag)


_G_CH = 128
_G_NCH = R // _G_CH // NW


def _sc_mesh():
    return plsc.VectorSubcoreMesh(core_axis_name="c", subcore_axis_name="s")


_G_NS = 4
_G_SB = _G_CH // _G_NS


def _sc_gather_body(hs_hbm, tok_hbm, nrow_hbm, out_hbm, nrow_v, idx_v,
                    b0, b1, b2, b3, s0, s1, s2, s3):
    wid = lax.axis_index("s") * SC_CORES + lax.axis_index("c")
    pltpu.sync_copy(nrow_hbm, nrow_v)
    n = nrow_v[...][0]
    bufs = (b0, b1, b2, b3)
    sems = (s0, s1, s2, s3)
    for c in range(_G_NCH):
        o = (wid + NW * c) * _G_CH

        @pl.when(o < n)
        def _():
            pltpu.sync_copy(tok_hbm.at[pl.ds(o, _G_CH)], idx_v)
            cps = [
                pltpu.async_copy(
                    hs_hbm.at[idx_v.at[pl.ds(b * _G_SB, _G_SB)]],
                    bufs[b], sems[b])
                for b in range(_G_NS)
            ]
            for b in range(_G_NS):
                cps[b].wait()
                pltpu.sync_copy(bufs[b],
                                out_hbm.at[pl.ds(o + b * _G_SB, _G_SB)])


def _sc_gather(hidden_states, row_tok, n_rows):
    return pl.kernel(
        _sc_gather_body,
        out_type=jax.ShapeDtypeStruct((R, H), jnp.float32),
        mesh=_sc_mesh(),
        scratch_types=[
            pltpu.VMEM((16,), jnp.int32),
            pltpu.VMEM((_G_CH,), jnp.int32),
            pltpu.VMEM((_G_SB, H), jnp.float32),
            pltpu.VMEM((_G_SB, H), jnp.float32),
            pltpu.VMEM((_G_SB, H), jnp.float32),
            pltpu.VMEM((_G_SB, H), jnp.float32),
            pltpu.SemaphoreType.DMA,
            pltpu.SemaphoreType.DMA,
            pltpu.SemaphoreType.DMA,
            pltpu.SemaphoreType.DMA,
        ],
    )(hidden_states, row_tok, n_rows)


def _ffn_body(te_ref, tc_ref, tb_ref, rt_ref, ne_ref, pa_ref, ns_ref, nf_ref,
              hs_ref, w1_hbm, w2_hbm, rw_ref, y_ref, x2_s, w1b, w2b, s1, s2):
    i = pl.program_id(0)
    pp = pa_ref[i]

    def w_copies(src_e, dst_p):
        return (
            pltpu.make_async_copy(w1_hbm.at[src_e], w1b.at[dst_p], s1.at[dst_p]),
            pltpu.make_async_copy(w2_hbm.at[src_e], w2b.at[dst_p], s2.at[dst_p]),
        )

    @pl.when(i == 0)
    def _():
        for cp in w_copies(te_ref[0], 0):
            cp.start()
        for j in range(T):
            t0 = rt_ref[j]
            x2_s[0, pl.ds(j, 1), :] = hs_ref[pl.ds(t0, 1), :]

    @pl.when(ne_ref[i] > 0)
    def _():
        for cp in w_copies(te_ref[i], pp):
            cp.wait()

        @pl.when(nf_ref[i] > 0)
        def _():
            for cp in w_copies(ns_ref[i], 1 - pp):
                cp.start()

    @pl.when(tc_ref[i] > 0)
    def _():
        par = lax.rem(i, 2)
        nb = 1 - par
        nxt = jnp.minimum(i + 1, NT - 1)
        for j in range(T):
            t = rt_ref[nxt * T + j]
            x2_s[nb, pl.ds(j, 1), :] = hs_ref[pl.ds(t, 1), :]

        x = x2_s[par]
        gu = lax.dot_general(x, w1b[pp], (((1,), (1,)), ((), ())),
                             preferred_element_type=jnp.float32)
        g = gu[:, :F]
        u = gu[:, F:]
        h = g * (1.0 / (1.0 + jnp.exp(-g))) * u
        y = lax.dot_general(h, w2b[pp], (((1,), (1,)), ((), ())),
                            preferred_element_type=jnp.float32)
        y_ref[...] = y * rw_ref[0, :, 0:1]


def _ffn(hidden_states, w1, w2, rw_brd, tile_expert, tile_count, tile_blk,
         row_tok, new_e, par_t, nxt_src, nxt_flag):
    grid_spec = pltpu.PrefetchScalarGridSpec(
        num_scalar_prefetch=8,
        grid=(NT,),
        in_specs=[
            pl.BlockSpec((S, H), lambda i, *_: (0, 0)),
            pl.BlockSpec(memory_space=pl.ANY),
            pl.BlockSpec(memory_space=pl.ANY),
            pl.BlockSpec((1, T, 128), lambda i, te, tc, tb, *_: (tb[i], 0, 0)),
        ],
        out_specs=pl.BlockSpec((T, H), lambda i, te, tc, tb, *_: (tb[i], 0)),
        scratch_shapes=[
            pltpu.VMEM((2, T, H), jnp.float32),
            pltpu.VMEM((2, 2 * F, H), jnp.float32),
            pltpu.VMEM((2, H, F), jnp.float32),
            pltpu.SemaphoreType.DMA((2,)),
            pltpu.SemaphoreType.DMA((2,)),
        ],
    )
    return pl.pallas_call(
        _ffn_body,
        grid_spec=grid_spec,
        out_shape=jax.ShapeDtypeStruct((R, H), jnp.float32),
        compiler_params=pltpu.CompilerParams(
            dimension_semantics=("arbitrary",)),
    )(tile_expert, tile_count, tile_blk, row_tok, new_e, par_t, nxt_src,
      nxt_flag, hidden_states, w1, w2, rw_brd)


_C_PER_W = S // NW
_C_HALF = _C_PER_W // 2
_LANES = 16


def _sc_combine_body(y_hbm, p0_hbm, p1_hbm, out_hbm, i0_v, i1_v, r0_v, r1_v,
                     s0, s1, s2, s3):
    wid = lax.axis_index("s") * SC_CORES + lax.axis_index("c")
    o = wid * _C_PER_W
    pltpu.sync_copy(p0_hbm.at[pl.ds(o, _C_PER_W)], i0_v)
    pltpu.sync_copy(p1_hbm.at[pl.ds(o, _C_PER_W)], i1_v)
    cps = [
        pltpu.async_copy(y_hbm.at[i0_v.at[pl.ds(0, _C_HALF)]],
                         r0_v.at[pl.ds(0, _C_HALF)], s0),
        pltpu.async_copy(y_hbm.at[i0_v.at[pl.ds(_C_HALF, _C_HALF)]],
                         r0_v.at[pl.ds(_C_HALF, _C_HALF)], s1),
        pltpu.async_copy(y_hbm.at[i1_v.at[pl.ds(0, _C_HALF)]],
                         r1_v.at[pl.ds(0, _C_HALF)], s2),
        pltpu.async_copy(y_hbm.at[i1_v.at[pl.ds(_C_HALF, _C_HALF)]],
                         r1_v.at[pl.ds(_C_HALF, _C_HALF)], s3),
    ]
    for cp in cps:
        cp.wait()

    def row_add(i, _):
        def lane_add(j, _):
            sl = pl.ds(j * _LANES, _LANES)
            r0_v[i, sl] = r0_v[i, sl] + r1_v[i, sl]
            return 0
        return lax.fori_loop(0, H // _LANES, lane_add, 0)

    lax.fori_loop(0, _C_PER_W, row_add, 0)
    pltpu.sync_copy(r0_v, out_hbm.at[pl.ds(o, _C_PER_W)])


def _sc_combine(y, p0, p1):
    return pl.kernel(
        _sc_combine_body,
        out_type=jax.ShapeDtypeStruct((S, H), jnp.float32),
        mesh=_sc_mesh(),
        scratch_types=[
            pltpu.VMEM((_C_PER_W,), jnp.int32),
            pltpu.VMEM((_C_PER_W,), jnp.int32),
            pltpu.VMEM((_C_PER_W, H), jnp.float32),
            pltpu.VMEM((_C_PER_W, H), jnp.float32),
            pltpu.SemaphoreType.DMA,
            pltpu.SemaphoreType.DMA,
            pltpu.SemaphoreType.DMA,
            pltpu.SemaphoreType.DMA,
        ],
    )(y, p0, p1)


def kernel(hidden_states, router_logits, w1, w2):
    idx, wts = _routing(router_logits)
    (row_tok, rw_brd, p0, p1, tile_expert, tile_count, tile_blk, n_rows,
     new_e, par_t, nxt_src, nxt_flag) = _plan(idx, wts)
    y = _ffn(hidden_states, w1, w2, rw_brd, tile_expert, tile_count,
             tile_blk, row_tok, new_e, par_t, nxt_src, nxt_flag)
    return _sc_combine(y, p0, p1)


# speedup vs baseline: 1.2373x; 1.1173x over previous
import functools

import jax
import jax.numpy as jnp
from jax import lax
from jax.experimental import pallas as pl
from jax.experimental.pallas import tpu as pltpu
from jax.experimental.pallas import tpu_sc as plsc

E = 64
K = 2
H = 768
F = 512
S = 2048

T = 64
NT = S * K // T + E
R = NT * T

SC_CORES = 2
SC_SUBCORES = 16
NW = SC_CORES * SC_SUBCORES


def _routing_body(logits_ref, idx_ref, wts_ref):
    logits = logits_ref[...]
    m = jnp.max(logits, axis=1, keepdims=True)
    ex = jnp.exp(logits - m)
    probs = ex / jnp.sum(ex, axis=1, keepdims=True)
    iota = lax.broadcasted_iota(jnp.int32, (S, E), 1)
    m1 = jnp.max(probs, axis=1, keepdims=True)
    i1 = jnp.min(jnp.where(probs == m1, iota, E), axis=1, keepdims=True)
    probs2 = jnp.where(iota == i1, -1.0, probs)
    m2 = jnp.max(probs2, axis=1, keepdims=True)
    i2 = jnp.min(jnp.where(probs2 == m2, iota, E), axis=1, keepdims=True)
    ssum = m1 + m2
    idx_ref[:, 0:1] = i1
    idx_ref[:, 1:2] = i2
    wts_ref[:, 0:1] = m1 / ssum
    wts_ref[:, 1:2] = m2 / ssum


def _routing(router_logits):
    return pl.pallas_call(
        _routing_body,
        out_shape=(
            jax.ShapeDtypeStruct((S, 8), jnp.int32),
            jax.ShapeDtypeStruct((S, 8), jnp.float32),
        ),
    )(router_logits.astype(jnp.float32))


def _plan(idx, wts):
    flat_e = jnp.stack([idx[:, 0], idx[:, 1]], axis=1).reshape(-1)
    flat_w = jnp.stack([wts[:, 0], wts[:, 1]], axis=1).reshape(-1)
    onehot = (flat_e[:, None] == jnp.arange(E, dtype=jnp.int32)[None, :])
    ohf = onehot.astype(jnp.float32)
    cum = jnp.cumsum(ohf, axis=0)
    rank = (jnp.sum(ohf * cum, axis=1) - 1.0).astype(jnp.int32)
    counts = cum[-1].astype(jnp.int32)
    tiles_per_e = (counts + T - 1) // T
    tile_csum = jnp.cumsum(tiles_per_e)
    toff = tile_csum - tiles_per_e
    toff_e = (ohf @ toff.astype(jnp.float32)).astype(jnp.int32)
    padded_pos = toff_e * T + rank

    tok_ids = (jnp.arange(S * K, dtype=jnp.int32) // K)
    row_tok = jnp.zeros((R,), jnp.int32).at[padded_pos].set(tok_ids)
    row_w = jnp.zeros((R,), jnp.float32).at[padded_pos].set(flat_w)
    p0 = padded_pos[0::2]
    p1 = padded_pos[1::2]

    total_tiles = tile_csum[-1]
    tids = jnp.arange(NT, dtype=jnp.int32)
    raw_te = jnp.searchsorted(tile_csum, tids, side="right").astype(jnp.int32)
    last_used = jnp.max(jnp.where(counts > 0, jnp.arange(E), -1)).astype(jnp.int32)
    used = tids < total_tiles
    tile_expert = jnp.where(used, jnp.minimum(raw_te, E - 1), last_used)
    cnt = counts[tile_expert].astype(jnp.int32) - (tids - toff[tile_expert].astype(jnp.int32)) * T
    tile_count = jnp.where(used, jnp.clip(cnt, 0, T), 0).astype(jnp.int32)

    rw_brd = jnp.broadcast_to(row_w.reshape(NT, T, 1), (NT, T, 128))
    tile_blk = jnp.where(used, tids, jnp.maximum(total_tiles - 1, 0)).astype(jnp.int32)
    n_rows = jnp.broadcast_to((total_tiles * T).astype(jnp.int32), (16,))

    used_e = counts > 0
    n_used = jnp.sum(used_e.astype(jnp.int32))
    rank_e = jnp.cumsum(used_e.astype(jnp.int32)) - 1
    eidx_t = rank_e[tile_expert]
    new_e = (used & (tids == toff[tile_expert].astype(jnp.int32))).astype(jnp.int32)
    par_t = (eidx_t % 3).astype(jnp.int32)
    exp_of_rank = jnp.searchsorted(
        jnp.cumsum(used_e.astype(jnp.int32)),
        jnp.arange(1, E + 1, dtype=jnp.int32), side="left").astype(jnp.int32)
    nxt_src = jnp.clip(exp_of_rank[jnp.clip(eidx_t + 2, 0, E - 1)], 0, E - 1)
    nxt_flag = (new_e.astype(bool) & (eidx_t + 2 < n_used)).astype(jnp.int32)

    nxt1_flag = jnp.broadcast_to((n_used > 1).astype(jnp.int32), (NT,))
    nxt1_src = jnp.broadcast_to(jnp.clip(exp_of_rank[1], 0, E - 1), (NT,))

    return (row_tok, rw_brd, p0, p1, tile_expert, tile_count, tile_blk,
            n_rows, new_e, par_t, nxt_src, nxt_flag, nxt1_flag, nxt1_src)


_G_CH = 128
_G_NCH = R // _G_CH // NW


def _sc_mesh():
    return plsc.VectorSubcoreMesh(core_axis_name="c", subcore_axis_name="s")


_G_NS = 4
_G_SB = _G_CH // _G_NS


def _sc_gather_body(hs_hbm, tok_hbm, nrow_hbm, out_hbm, nrow_v, idx_v,
                    b0, b1, b2, b3, s0, s1, s2, s3):
    wid = lax.axis_index("s") * SC_CORES + lax.axis_index("c")
    pltpu.sync_copy(nrow_hbm, nrow_v)
    n = nrow_v[...][0]
    bufs = (b0, b1, b2, b3)
    sems = (s0, s1, s2, s3)
    for c in range(_G_NCH):
        o = (wid + NW * c) * _G_CH

        @pl.when(o < n)
        def _():
            pltpu.sync_copy(tok_hbm.at[pl.ds(o, _G_CH)], idx_v)
            cps = [
                pltpu.async_copy(
                    hs_hbm.at[idx_v.at[pl.ds(b * _G_SB, _G_SB)]],
                    bufs[b], sems[b])
                for b in range(_G_NS)
            ]
            for b in range(_G_NS):
                cps[b].wait()
                pltpu.sync_copy(bufs[b],
                                out_hbm.at[pl.ds(o + b * _G_SB, _G_SB)])


def _sc_gather(hidden_states, row_tok, n_rows):
    return pl.kernel(
        _sc_gather_body,
        out_type=jax.ShapeDtypeStruct((R, H), jnp.float32),
        mesh=_sc_mesh(),
        scratch_types=[
            pltpu.VMEM((16,), jnp.int32),
            pltpu.VMEM((_G_CH,), jnp.int32),
            pltpu.VMEM((_G_SB, H), jnp.float32),
            pltpu.VMEM((_G_SB, H), jnp.float32),
            pltpu.VMEM((_G_SB, H), jnp.float32),
            pltpu.VMEM((_G_SB, H), jnp.float32),
            pltpu.SemaphoreType.DMA,
            pltpu.SemaphoreType.DMA,
            pltpu.SemaphoreType.DMA,
            pltpu.SemaphoreType.DMA,
        ],
    )(hidden_states, row_tok, n_rows)


def _ffn_body(te_ref, tc_ref, tb_ref, rt_ref, ne_ref, pa_ref, ns_ref, nf_ref,
              ne2_ref, ns2_ref, hs_ref, w1_hbm, w2_hbm, rw_ref, y_ref, x2_s,
              w1b, w2b, s1, s2):
    i = pl.program_id(0)
    pp = pa_ref[i]

    def w_copies(src_e, dst_p):
        return (
            pltpu.make_async_copy(w1_hbm.at[src_e], w1b.at[dst_p], s1.at[dst_p]),
            pltpu.make_async_copy(w2_hbm.at[src_e], w2b.at[dst_p], s2.at[dst_p]),
        )

    @pl.when(i == 0)
    def _():
        for cp in w_copies(te_ref[0], 0):
            cp.start()

        @pl.when(ne2_ref[0] > 0)
        def _():
            for cp in w_copies(ns2_ref[0], 1):
                cp.start()

        for j in range(T):
            t0 = rt_ref[j]
            x2_s[0, pl.ds(j, 1), :] = hs_ref[pl.ds(t0, 1), :]

    @pl.when(ne_ref[i] > 0)
    def _():
        for cp in w_copies(te_ref[i], pp):
            cp.wait()

        @pl.when(nf_ref[i] > 0)
        def _():
            nslot = lax.rem(pp + 2, 3)
            for cp in w_copies(ns_ref[i], nslot):
                cp.start()

    @pl.when(tc_ref[i] > 0)
    def _():
        par = lax.rem(i, 2)
        nb = 1 - par
        nxt = jnp.minimum(i + 1, NT - 1)
        for j in range(T):
            t = rt_ref[nxt * T + j]
            x2_s[nb, pl.ds(j, 1), :] = hs_ref[pl.ds(t, 1), :]

        x = x2_s[par]
        gu = lax.dot_general(x, w1b[pp], (((1,), (1,)), ((), ())),
                             preferred_element_type=jnp.float32)
        g = gu[:, :F]
        u = gu[:, F:]
        h = g * (1.0 / (1.0 + jnp.exp(-g))) * u
        y = lax.dot_general(h, w2b[pp], (((1,), (1,)), ((), ())),
                            preferred_element_type=jnp.float32)
        y_ref[...] = y * rw_ref[0, :, 0:1]


def _ffn(hidden_states, w1, w2, rw_brd, tile_expert, tile_count, tile_blk,
         row_tok, new_e, par_t, nxt_src, nxt_flag, nxt1_flag, nxt1_src):
    grid_spec = pltpu.PrefetchScalarGridSpec(
        num_scalar_prefetch=10,
        grid=(NT,),
        in_specs=[
            pl.BlockSpec((S, H), lambda i, *_: (0, 0)),
            pl.BlockSpec(memory_space=pl.ANY),
            pl.BlockSpec(memory_space=pl.ANY),
            pl.BlockSpec((1, T, 128), lambda i, te, tc, tb, *_: (tb[i], 0, 0)),
        ],
        out_specs=pl.BlockSpec((T, H), lambda i, te, tc, tb, *_: (tb[i], 0)),
        scratch_shapes=[
            pltpu.VMEM((2, T, H), jnp.float32),
            pltpu.VMEM((3, 2 * F, H), jnp.float32),
            pltpu.VMEM((3, H, F), jnp.float32),
            pltpu.SemaphoreType.DMA((3,)),
            pltpu.SemaphoreType.DMA((3,)),
        ],
    )
    return pl.pallas_call(
        _ffn_body,
        grid_spec=grid_spec,
        out_shape=jax.ShapeDtypeStruct((R, H), jnp.float32),
        compiler_params=pltpu.CompilerParams(
            dimension_semantics=("arbitrary",)),
    )(tile_expert, tile_count, tile_blk, row_tok, new_e, par_t, nxt_src,
      nxt_flag, nxt1_flag, nxt1_src, hidden_states, w1, w2, rw_brd)


_C_PER_W = S // NW
_C_HALF = _C_PER_W // 2
_LANES = 16


def _sc_combine_body(y_hbm, p0_hbm, p1_hbm, out_hbm, i0_v, i1_v, r0_v, r1_v,
                     s0, s1, s2, s3):
    wid = lax.axis_index("s") * SC_CORES + lax.axis_index("c")
    o = wid * _C_PER_W
    pltpu.sync_copy(p0_hbm.at[pl.ds(o, _C_PER_W)], i0_v)
    pltpu.sync_copy(p1_hbm.at[pl.ds(o, _C_PER_W)], i1_v)
    cps = [
        pltpu.async_copy(y_hbm.at[i0_v.at[pl.ds(0, _C_HALF)]],
                         r0_v.at[pl.ds(0, _C_HALF)], s0),
        pltpu.async_copy(y_hbm.at[i0_v.at[pl.ds(_C_HALF, _C_HALF)]],
                         r0_v.at[pl.ds(_C_HALF, _C_HALF)], s1),
        pltpu.async_copy(y_hbm.at[i1_v.at[pl.ds(0, _C_HALF)]],
                         r1_v.at[pl.ds(0, _C_HALF)], s2),
        pltpu.async_copy(y_hbm.at[i1_v.at[pl.ds(_C_HALF, _C_HALF)]],
                         r1_v.at[pl.ds(_C_HALF, _C_HALF)], s3),
    ]
    for cp in cps:
        cp.wait()

    def row_add(i, _):
        def lane_add(j, _):
            sl = pl.ds(j * _LANES, _LANES)
            r0_v[i, sl] = r0_v[i, sl] + r1_v[i, sl]
            return 0
        return lax.fori_loop(0, H // _LANES, lane_add, 0)

    lax.fori_loop(0, _C_PER_W, row_add, 0)
    pltpu.sync_copy(r0_v, out_hbm.at[pl.ds(o, _C_PER_W)])


def _sc_combine(y, p0, p1):
    return pl.kernel(
        _sc_combine_body,
        out_type=jax.ShapeDtypeStruct((S, H), jnp.float32),
        mesh=_sc_mesh(),
        scratch_types=[
            pltpu.VMEM((_C_PER_W,), jnp.int32),
            pltpu.VMEM((_C_PER_W,), jnp.int32),
            pltpu.VMEM((_C_PER_W, H), jnp.float32),
            pltpu.VMEM((_C_PER_W, H), jnp.float32),
            pltpu.SemaphoreType.DMA,
            pltpu.SemaphoreType.DMA,
            pltpu.SemaphoreType.DMA,
            pltpu.SemaphoreType.DMA,
        ],
    )(y, p0, p1)


def kernel(hidden_states, router_logits, w1, w2):
    idx, wts = _routing(router_logits)
    (row_tok, rw_brd, p0, p1, tile_expert, tile_count, tile_blk, n_rows,
     new_e, par_t, nxt_src, nxt_flag, nxt1_flag, nxt1_src) = _plan(idx, wts)
    y = _ffn(hidden_states, w1, w2, rw_brd, tile_expert, tile_count,
             tile_blk, row_tok, new_e, par_t, nxt_src, nxt_flag, nxt1_flag,
             nxt1_src)
    return _sc_combine(y, p0, p1)
